# baseline (speedup 1.0000x reference)
import functools
import jax
import jax.numpy as jnp
from jax import lax
from jax.experimental import pallas as pl
from jax.experimental.pallas import tpu as pltpu
from jax.experimental.pallas import tpu_sc as plsc

MAXP = 4096
D = 1024
NC = 2
NS = 16
NW = NC * NS
CHUNK = 32


def _make_sc(nrows):
    rows_per_w = nrows // NW
    nchunk = rows_per_w // CHUNK
    cwords = CHUNK * D
    mesh = plsc.VectorSubcoreMesh(core_axis_name="c", subcore_axis_name="s")

    @functools.partial(
        pl.kernel,
        mesh=mesh,
        out_type=jax.ShapeDtypeStruct((nrows * D,), jnp.float32),
        scratch_types=[
            pltpu.VMEM((cwords,), jnp.float32),
            pltpu.VMEM((cwords,), jnp.float32),
        ],
    )
    def body(x_hbm, tab_hbm, out_hbm, bufx, buft):
        cid = lax.axis_index("c")
        sid = lax.axis_index("s")
        wid = sid * NC + cid
        rowbase = wid * rows_per_w
        sbase = lax.rem(rowbase, MAXP)
        for c in range(nchunk):
            r0 = (rowbase + c * CHUNK) * D
            t0 = (sbase + c * CHUNK) * D
            pltpu.sync_copy(x_hbm.at[pl.ds(r0, cwords)], bufx)
            pltpu.sync_copy(tab_hbm.at[pl.ds(t0, cwords)], buft)

            def addone(i, carry):
                o = i * 16
                bufx[pl.ds(o, 16)] = bufx[pl.ds(o, 16)] + buft[pl.ds(o, 16)]
                return carry

            lax.fori_loop(0, cwords // 16, addone, 0)
            pltpu.sync_copy(bufx, out_hbm.at[pl.ds(r0, cwords)])

    return body


def kernel(x, pos_table):
    b, s, d = x.shape
    x1 = x.reshape(b * s * d)
    t1 = pos_table.reshape(MAXP * D)
    out = _make_sc(b * s)(x1, t1)
    return out.reshape(b, s, d)


# speedup vs baseline: 1.4689x; 1.4689x over previous
import functools
import jax
import jax.numpy as jnp
from jax import lax
from jax.experimental import pallas as pl
from jax.experimental.pallas import tpu as pltpu
from jax.experimental.pallas import tpu_sc as plsc

MAXP = 4096
D = 1024
NC = 2
NS = 16
NW = NC * NS
CHUNK = 16


def _make_sc(nrows):
    rows_per_w = nrows // NW
    nchunk = rows_per_w // CHUNK
    cw = CHUNK * D
    mesh = plsc.VectorSubcoreMesh(core_axis_name="c", subcore_axis_name="s")

    @functools.partial(
        pl.kernel,
        mesh=mesh,
        out_type=jax.ShapeDtypeStruct((nrows * D,), jnp.float32),
        scratch_types=[
            pltpu.VMEM((2, cw), jnp.float32),
            pltpu.VMEM((2, cw), jnp.float32),
            pltpu.VMEM((2, cw), jnp.float32),
            pltpu.SemaphoreType.DMA((2,)),
            pltpu.SemaphoreType.DMA((2,)),
            pltpu.SemaphoreType.DMA((2,)),
        ],
    )
    def body(x_hbm, tab_hbm, out_hbm, bufx, buft, bufo, semx, semt, semo):
        cid = lax.axis_index("c")
        sid = lax.axis_index("s")
        wid = sid * NC + cid
        rowbase = wid * rows_per_w
        sbase = lax.rem(rowbase, MAXP)

        def start_in(c, slot):
            r0 = (rowbase + c * CHUNK) * D
            t0 = (sbase + c * CHUNK) * D
            pltpu.async_copy(x_hbm.at[pl.ds(r0, cw)], bufx.at[slot], semx.at[slot])
            pltpu.async_copy(tab_hbm.at[pl.ds(t0, cw)], buft.at[slot], semt.at[slot])

        start_in(0, 0)
        start_in(1, 1)
        for c in range(nchunk):
            cur = c % 2
            pltpu.make_async_copy(x_hbm.at[pl.ds(0, cw)], bufx.at[cur], semx.at[cur]).wait()
            pltpu.make_async_copy(tab_hbm.at[pl.ds(0, cw)], buft.at[cur], semt.at[cur]).wait()
            if c >= 2:
                pltpu.make_async_copy(bufo.at[cur], out_hbm.at[pl.ds(0, cw)], semo.at[cur]).wait()

            @plsc.parallel_loop(0, cw, 16 * 8)
            def _(i):
                for u in range(8):
                    o = i + u * 16
                    bufo[cur, pl.ds(o, 16)] = bufx[cur, pl.ds(o, 16)] + buft[cur, pl.ds(o, 16)]

            r0 = (rowbase + c * CHUNK) * D
            pltpu.async_copy(bufo.at[cur], out_hbm.at[pl.ds(r0, cw)], semo.at[cur])
            if c + 2 < nchunk:
                start_in(c + 2, cur)
        for cur in (nchunk % 2, (nchunk + 1) % 2):
            pltpu.make_async_copy(bufo.at[cur], out_hbm.at[pl.ds(0, cw)], semo.at[cur]).wait()

    return body


def kernel(x, pos_table):
    b, s, d = x.shape
    x1 = x.reshape(b * s * d)
    t1 = pos_table.reshape(MAXP * D)
    out = _make_sc(b * s)(x1, t1)
    return out.reshape(b, s, d)


# speedup vs baseline: 4.3231x; 2.9431x over previous
import functools
import jax
import jax.numpy as jnp
from jax import lax
from jax.experimental import pallas as pl
from jax.experimental.pallas import tpu as pltpu
from jax.experimental.pallas import tpu_sc as plsc

MAXP = 4096
D = 1024
NC = 2
NS = 16
NW = NC * NS
CHUNK = 16


def _make_sc(nrows):
    rows_per_w = nrows // NW
    nchunk = rows_per_w // CHUNK
    assert nchunk % 2 == 0 and nchunk >= 4
    mesh = plsc.VectorSubcoreMesh(core_axis_name="c", subcore_axis_name="s")

    @functools.partial(
        pl.kernel,
        mesh=mesh,
        out_type=jax.ShapeDtypeStruct((nrows, D), jnp.float32),
        scratch_types=[
            pltpu.VMEM((2, CHUNK, D), jnp.float32),
            pltpu.VMEM((2, CHUNK, D), jnp.float32),
            pltpu.VMEM((2, CHUNK, D), jnp.float32),
            pltpu.SemaphoreType.DMA((2,)),
            pltpu.SemaphoreType.DMA((2,)),
            pltpu.SemaphoreType.DMA((2,)),
        ],
    )
    def body(x_hbm, tab_hbm, out_hbm, bufx, buft, bufo, semx, semt, semo):
        cid = lax.axis_index("c")
        sid = lax.axis_index("s")
        wid = sid * NC + cid
        rowbase = wid * rows_per_w
        sbase = lax.rem(rowbase, MAXP)

        def start_in(c, slot):
            r0 = rowbase + c * CHUNK
            t0 = sbase + c * CHUNK
            pltpu.async_copy(x_hbm.at[pl.ds(r0, CHUNK)], bufx.at[slot], semx.at[slot])
            pltpu.async_copy(tab_hbm.at[pl.ds(t0, CHUNK)], buft.at[slot], semt.at[slot])

        def wait_in(slot):
            pltpu.make_async_copy(x_hbm.at[pl.ds(0, CHUNK)], bufx.at[slot], semx.at[slot]).wait()
            pltpu.make_async_copy(tab_hbm.at[pl.ds(0, CHUNK)], buft.at[slot], semt.at[slot]).wait()

        def wait_out(slot):
            pltpu.make_async_copy(bufo.at[slot], out_hbm.at[pl.ds(0, CHUNK)], semo.at[slot]).wait()

        def add_chunk(slot):
            @plsc.parallel_loop(0, CHUNK * D, 16 * 8)
            def _(i):
                r = lax.shift_right_logical(i, 10)
                o = pl.multiple_of(lax.bitwise_and(i, D - 1), 16 * 8)
                for u in range(8):
                    bufo[slot, r, pl.ds(o + u * 16, 16)] = (
                        bufx[slot, r, pl.ds(o + u * 16, 16)]
                        + buft[slot, r, pl.ds(o + u * 16, 16)]
                    )

        def store_out(c, slot):
            r0 = rowbase + c * CHUNK
            pltpu.async_copy(bufo.at[slot], out_hbm.at[pl.ds(r0, CHUNK)], semo.at[slot])

        start_in(0, 0)
        start_in(1, 1)
        wait_in(0)
        add_chunk(0)
        store_out(0, 0)
        start_in(2, 0)
        wait_in(1)
        add_chunk(1)
        store_out(1, 1)
        start_in(3, 1)

        def pair(g, carry):
            for k in (0, 1):
                c = g * 2 + k
                wait_in(k)
                wait_out(k)
                add_chunk(k)
                store_out(c, k)

                @pl.when(c + 2 < nchunk)
                def _():
                    start_in(c + 2, k)

            return carry

        lax.fori_loop(1, nchunk // 2, pair, 0)
        wait_out(0)
        wait_out(1)

    return body


def kernel(x, pos_table):
    b, s, d = x.shape
    x2 = x.reshape(b * s, d)
    out = _make_sc(b * s)(x2, pos_table)
    return out.reshape(b, s, d)
